# Initial kernel scaffold; baseline (speedup 1.0000x reference)
import jax
import jax.numpy as jnp
from jax.experimental import pallas as pl


def kernel(x, edge_index, edge_attr, W1_l, b1_l, W1_r, W2_l, b2_l, W2_r):
    raise NotImplementedError("write your pallas kernel here")


# speedup vs baseline: 5.7146x; 5.7146x over previous
import functools

import jax
import jax.numpy as jnp
from jax import lax
from jax.experimental import pallas as pl
from jax.experimental.pallas import tpu as pltpu
from jax.experimental.pallas import tpu_sc as plsc

N_NODES = 10000
N_EDGES = 320000
IN_DIM = 128
HID_DIM = 256
OUT_DIM = 64

NC, NS = 2, 16
NW = NC * NS
EPT = N_EDGES // NW
CHUNK = 80
NCHUNK = EPT // CHUNK
NPAD = 10240
RPT = NPAD // NS
AUG = IN_DIM + 16

_mesh = plsc.VectorSubcoreMesh(core_axis_name="c", subcore_axis_name="s")


def _zero_rows(buf, n_rows, width):
    zero = jnp.zeros((16,), jnp.float32)

    def body(i, _):
        for j in range(width // 16):
            buf[i, pl.ds(j * 16, 16)] = zero
        return 0

    lax.fori_loop(0, n_rows, body, 0)


def _clear_slice(acc, base_r, zsrc):
    for k in range(RPT // CHUNK):
        pltpu.sync_copy(zsrc, acc.at[pl.ds(base_r + k * CHUNK, CHUNK)])


@functools.partial(
    pl.kernel,
    out_type=jax.ShapeDtypeStruct((NC, NPAD, AUG), jnp.float32),
    mesh=_mesh,
    compiler_params=pltpu.CompilerParams(use_tc_tiling_on_sc=False),
    scratch_types=[
        pltpu.VMEM((CHUNK,), jnp.int32),
        pltpu.VMEM((CHUNK,), jnp.int32),
        pltpu.VMEM((CHUNK,), jnp.float32),
        pltpu.VMEM((CHUNK, IN_DIM), jnp.float32),
        pltpu.VMEM((CHUNK, AUG), jnp.float32),
        pltpu.VMEM_SHARED((NPAD, AUG), jnp.float32),
        pltpu.SemaphoreType.DMA,
    ],
)
def _sc_pass1(esrc_hbm, edst_hbm, ea_hbm, x_hbm, out_hbm,
              src_v, dst_v, w_v, rows_v, msg_v, acc_sh, sem):
    c = lax.axis_index("c")
    s = lax.axis_index("s")
    base_r = s * RPT

    _zero_rows(msg_v, CHUNK, AUG)
    _clear_slice(acc_sh, base_r, msg_v)

    one = jnp.ones((16,), jnp.float32)

    def set_ones(i, _):
        msg_v[i, pl.ds(IN_DIM, 16)] = one
        return 0

    lax.fori_loop(0, CHUNK, set_ones, 0)
    plsc.subcore_barrier()

    ebase = (c * NS + s) * EPT

    def chunk_body(k, _):
        off = ebase + k * CHUNK
        pltpu.sync_copy(esrc_hbm.at[pl.ds(off, CHUNK)], src_v)
        pltpu.sync_copy(edst_hbm.at[pl.ds(off, CHUNK)], dst_v)
        pltpu.sync_copy(ea_hbm.at[pl.ds(off, CHUNK)], w_v)
        pltpu.async_copy(x_hbm.at[src_v], rows_v, sem).wait()

        def scale(i, _):
            wvec = w_v[pl.ds(i * 16, 16)]
            for l in range(16):
                we = wvec[l]
                row = i * 16 + l
                for j in range(IN_DIM // 16):
                    msg_v[row, pl.ds(j * 16, 16)] = (
                        rows_v[row, pl.ds(j * 16, 16)] * we)
            return 0

        lax.fori_loop(0, CHUNK // 16, scale, 0)
        pltpu.sync_copy(msg_v, acc_sh.at[dst_v], add=True)
        return 0

    lax.fori_loop(0, NCHUNK, chunk_body, 0)
    plsc.subcore_barrier()
    pltpu.sync_copy(acc_sh.at[pl.ds(base_r, RPT)],
                    out_hbm.at[c, pl.ds(base_r, RPT)])


@functools.partial(
    pl.kernel,
    out_type=jax.ShapeDtypeStruct((NC, NPAD, OUT_DIM), jnp.float32),
    mesh=_mesh,
    compiler_params=pltpu.CompilerParams(use_tc_tiling_on_sc=False),
    scratch_types=[
        pltpu.VMEM((CHUNK,), jnp.int32),
        pltpu.VMEM((CHUNK,), jnp.int32),
        pltpu.VMEM((CHUNK,), jnp.float32),
        pltpu.VMEM((CHUNK, OUT_DIM), jnp.float32),
        pltpu.VMEM((CHUNK, OUT_DIM), jnp.float32),
        pltpu.VMEM_SHARED((NPAD, OUT_DIM), jnp.float32),
        pltpu.SemaphoreType.DMA,
    ],
)
def _sc_pass2(esrc_hbm, edst_hbm, ea_hbm, p_hbm, out_hbm,
              src_v, dst_v, w_v, rows_v, msg_v, acc_sh, sem):
    c = lax.axis_index("c")
    s = lax.axis_index("s")
    base_r = s * RPT

    _zero_rows(msg_v, CHUNK, OUT_DIM)
    _clear_slice(acc_sh, base_r, msg_v)
    plsc.subcore_barrier()

    ebase = (c * NS + s) * EPT

    def chunk_body(k, _):
        off = ebase + k * CHUNK
        pltpu.sync_copy(esrc_hbm.at[pl.ds(off, CHUNK)], src_v)
        pltpu.sync_copy(edst_hbm.at[pl.ds(off, CHUNK)], dst_v)
        pltpu.sync_copy(ea_hbm.at[pl.ds(off, CHUNK)], w_v)
        pltpu.async_copy(p_hbm.at[src_v], rows_v, sem).wait()

        def scale(i, _):
            wvec = w_v[pl.ds(i * 16, 16)]
            for l in range(16):
                we = wvec[l]
                row = i * 16 + l
                for j in range(OUT_DIM // 16):
                    msg_v[row, pl.ds(j * 16, 16)] = (
                        rows_v[row, pl.ds(j * 16, 16)] * we)
            return 0

        lax.fori_loop(0, CHUNK // 16, scale, 0)
        pltpu.sync_copy(msg_v, acc_sh.at[dst_v], add=True)
        return 0

    lax.fori_loop(0, NCHUNK, chunk_body, 0)
    plsc.subcore_barrier()
    pltpu.sync_copy(acc_sh.at[pl.ds(base_r, RPT)],
                    out_hbm.at[c, pl.ds(base_r, RPT)])


BLK = 1000


def _tc1_body(acc_ref, x_ref, w1l_ref, b1l_ref, w1r_ref, w2l_ref, b2l_ref,
              w2r_ref, p_ref, r_ref):
    acc = acc_ref[0] + acc_ref[1]
    x = x_ref[...]
    agg = acc[:, :IN_DIM] + x
    cnt = acc[:, IN_DIM] + 1.0
    agg = agg / cnt[:, None]
    h = agg @ w1l_ref[...] + b1l_ref[...] + x @ w1r_ref[...]
    h = jnp.maximum(h, 0.0)
    p_ref[...] = h @ w2l_ref[...]
    r_ref[...] = h @ w2r_ref[...] + b2l_ref[...]


def _tc2_body(acc2_ref, p_ref, r_ref, accc_ref, out_ref):
    accc = accc_ref[...]
    cnt = accc[0, :, IN_DIM] + accc[1, :, IN_DIM] + 1.0
    g2 = acc2_ref[0] + acc2_ref[1] + p_ref[...]
    z = g2 / cnt[:, None] + r_ref[...]
    m = jnp.max(z, axis=1, keepdims=True)
    e = jnp.exp(z - m)
    out_ref[...] = (z - m) - jnp.log(jnp.sum(e, axis=1, keepdims=True))


def kernel(x, edge_index, edge_attr, W1_l, b1_l, W1_r, W2_l, b2_l, W2_r):
    esrc = edge_index[0]
    edst = edge_index[1]
    acc1 = _sc_pass1(esrc, edst, edge_attr, x)

    nblk = N_NODES // BLK
    full = lambda *shape: pl.BlockSpec(shape, lambda i: (0,) * len(shape))
    p, r = pl.pallas_call(
        _tc1_body,
        grid=(nblk,),
        in_specs=[
            pl.BlockSpec((NC, BLK, AUG), lambda i: (0, i, 0)),
            pl.BlockSpec((BLK, IN_DIM), lambda i: (i, 0)),
            full(IN_DIM, HID_DIM),
            full(1, HID_DIM),
            full(IN_DIM, HID_DIM),
            full(HID_DIM, OUT_DIM),
            full(1, OUT_DIM),
            full(HID_DIM, OUT_DIM),
        ],
        out_specs=[
            pl.BlockSpec((BLK, OUT_DIM), lambda i: (i, 0)),
            pl.BlockSpec((BLK, OUT_DIM), lambda i: (i, 0)),
        ],
        out_shape=[
            jax.ShapeDtypeStruct((N_NODES, OUT_DIM), jnp.float32),
            jax.ShapeDtypeStruct((N_NODES, OUT_DIM), jnp.float32),
        ],
    )(acc1, x, W1_l, b1_l.reshape(1, HID_DIM), W1_r, W2_l,
      b2_l.reshape(1, OUT_DIM), W2_r)

    acc2 = _sc_pass2(esrc, edst, edge_attr, p)

    out = pl.pallas_call(
        _tc2_body,
        grid=(nblk,),
        in_specs=[
            pl.BlockSpec((NC, BLK, OUT_DIM), lambda i: (0, i, 0)),
            pl.BlockSpec((BLK, OUT_DIM), lambda i: (i, 0)),
            pl.BlockSpec((BLK, OUT_DIM), lambda i: (i, 0)),
            pl.BlockSpec((NC, BLK, AUG), lambda i: (0, i, 0)),
        ],
        out_specs=pl.BlockSpec((BLK, OUT_DIM), lambda i: (i, 0)),
        out_shape=jax.ShapeDtypeStruct((N_NODES, OUT_DIM), jnp.float32),
    )(acc2, p, r, acc1)

    return out


# speedup vs baseline: 10.6707x; 1.8673x over previous
import functools

import jax
import jax.numpy as jnp
from jax import lax
from jax.experimental import pallas as pl
from jax.experimental.pallas import tpu as pltpu
from jax.experimental.pallas import tpu_sc as plsc

N_NODES = 10000
N_EDGES = 320000
IN_DIM = 128
HID_DIM = 256
OUT_DIM = 64

NC, NS = 2, 16
NW = NC * NS
CHUNK = 64
NROWS = N_EDGES // CHUNK
CPT = NROWS // NW
XTRA = NROWS - CPT * NW
NPAD = 10112
RPT = NPAD // NS
AUG = IN_DIM + 16

_mesh = plsc.VectorSubcoreMesh(core_axis_name="c", subcore_axis_name="s")


def _make_sc_pass(feat, aug):
    outw = feat + aug

    @functools.partial(
        pl.kernel,
        out_type=jax.ShapeDtypeStruct((NC, NPAD, outw), jnp.float32),
        mesh=_mesh,
        compiler_params=pltpu.CompilerParams(use_tc_tiling_on_sc=False, needs_layout_passes=False),
        scratch_types=[
            pltpu.VMEM((3, CHUNK), jnp.int32),
            pltpu.VMEM((3, CHUNK), jnp.int32),
            pltpu.VMEM((3, CHUNK), jnp.int32),
            pltpu.VMEM((3, CHUNK), jnp.int32),
            pltpu.VMEM((CHUNK, feat), jnp.float32),
            pltpu.VMEM((CHUNK, feat), jnp.float32),
            pltpu.VMEM((CHUNK, outw), jnp.float32),
            pltpu.VMEM((CHUNK, outw), jnp.float32),
            pltpu.VMEM_SHARED((NPAD, outw), jnp.float32),
            pltpu.SemaphoreType.DMA,
            pltpu.SemaphoreType.DMA,
            pltpu.SemaphoreType.DMA,
            pltpu.SemaphoreType.DMA,
            pltpu.SemaphoreType.DMA,
            pltpu.SemaphoreType.DMA,
            pltpu.SemaphoreType.DMA,
            pltpu.SemaphoreType.DMA,
        ],
    )
    def sc_pass(ed_hbm, tab_hbm, out_hbm,
                i0, i1, i2, i3, r0, r1, m0, m1, acc_sh,
                is0, is1, is2, is3, g0, g1, s0, s1):
        c = lax.axis_index("c")
        s = lax.axis_index("s")
        base_r = s * RPT
        ibuf, isem = (i0, i1, i2, i3), (is0, is1, is2, is3)
        rows, msgs, gs, ss = (r0, r1), (m0, m1), (g0, g1), (s0, s1)

        wid = c * NS + s
        start = wid * CPT + jnp.maximum(wid - (NW - XTRA), 0)
        has_xtra = wid >= NW - XTRA

        zero = jnp.zeros((16,), jnp.float32)
        one = jnp.ones((16,), jnp.float32)

        def zbody(i, _):
            for j in range(outw // 16):
                m0[i, pl.ds(j * 16, 16)] = zero
            return 0

        lax.fori_loop(0, CHUNK, zbody, 0)
        for k in range(RPT // CHUNK):
            pltpu.sync_copy(m0, acc_sh.at[pl.ds(base_r + k * CHUNK, CHUNK)])
        rem = RPT % CHUNK
        if rem:
            pltpu.sync_copy(
                m0.at[pl.ds(0, rem)],
                acc_sh.at[pl.ds(base_r + (RPT // CHUNK) * CHUNK, rem)])
        if aug:

            def obody(i, _):
                m0[i, pl.ds(feat, 16)] = one
                m1[i, pl.ds(feat, 16)] = one
                return 0

            lax.fori_loop(0, CHUNK, obody, 0)
        plsc.subcore_barrier()

        def idx_row(k):
            return jnp.minimum(start + k, NROWS - 1)

        def scale(j, b):
            def sbody(t, _):
                wvec = plsc.bitcast(ibuf[j][2, pl.ds(t * 16, 16)], jnp.float32)
                for l in range(16):
                    we = wvec[l]
                    row = t * 16 + l
                    for v in range(feat // 16):
                        msgs[b][row, pl.ds(v * 16, 16)] = (
                            rows[b][row, pl.ds(v * 16, 16)] * we)
                return 0

            lax.fori_loop(0, CHUNK // 16, sbody, 0)

        pltpu.async_copy(ed_hbm.at[idx_row(0)], ibuf[0], isem[0])
        pltpu.async_copy(ed_hbm.at[idx_row(1)], ibuf[1], isem[1])
        pltpu.make_async_copy(ed_hbm.at[0], ibuf[0], isem[0]).wait()
        pltpu.async_copy(tab_hbm.at[ibuf[0].at[0]], rows[0], gs[0])

        def body(i, _):
            for b in range(4):
                b2, nb2 = b % 2, (b + 1) % 2
                j2, j1 = (b + 2) % 4, (b + 1) % 4
                k = i * 4 + b
                if b < 2:
                    @pl.when(i > 0)
                    def _():
                        pltpu.make_async_copy(
                            msgs[b2], acc_sh.at[ibuf[b].at[1]], ss[b2]).wait()
                else:
                    pltpu.make_async_copy(
                        msgs[b2], acc_sh.at[ibuf[b].at[1]], ss[b2]).wait()
                pltpu.async_copy(ed_hbm.at[idx_row(k + 2)], ibuf[j2],
                                 isem[j2])
                pltpu.make_async_copy(ed_hbm.at[0], ibuf[j1], isem[j1]).wait()
                pltpu.async_copy(tab_hbm.at[ibuf[j1].at[0]], rows[nb2],
                                 gs[nb2])
                pltpu.make_async_copy(tab_hbm.at[ibuf[b].at[0]], rows[b2],
                                      gs[b2]).wait()
                scale(b, b2)
                pltpu.async_copy(msgs[b2], acc_sh.at[ibuf[b].at[1]], ss[b2],
                                 add=True)
            return 0

        lax.fori_loop(0, CPT // 4, body, 0)

        pltpu.make_async_copy(ed_hbm.at[0], ibuf[1], isem[1]).wait()
        pltpu.make_async_copy(tab_hbm.at[ibuf[0].at[0]], rows[0], gs[0]).wait()
        pltpu.make_async_copy(msgs[0], acc_sh.at[ibuf[0].at[1]], ss[0]).wait()
        pltpu.make_async_copy(msgs[1], acc_sh.at[ibuf[1].at[1]], ss[1]).wait()

        @pl.when(has_xtra)
        def _():
            scale(0, 0)
            pltpu.sync_copy(msgs[0], acc_sh.at[ibuf[0].at[1]], add=True)

        plsc.subcore_barrier()
        pltpu.sync_copy(acc_sh.at[pl.ds(base_r, RPT)],
                        out_hbm.at[c, pl.ds(base_r, RPT)])

    return sc_pass


_sc_pass1 = _make_sc_pass(IN_DIM, 16)
_sc_pass2 = _make_sc_pass(OUT_DIM, 0)


BLK = 1000


def _tc1_body(acc_ref, x_ref, w1l_ref, b1l_ref, w1r_ref, w2l_ref, b2l_ref,
              w2r_ref, p_ref, r_ref):
    acc = acc_ref[0] + acc_ref[1]
    x = x_ref[...]
    agg = acc[:, :IN_DIM] + x
    cnt = acc[:, IN_DIM] + 1.0
    agg = agg / cnt[:, None]
    h = agg @ w1l_ref[...] + b1l_ref[...] + x @ w1r_ref[...]
    h = jnp.maximum(h, 0.0)
    p_ref[...] = h @ w2l_ref[...]
    r_ref[...] = h @ w2r_ref[...] + b2l_ref[...]


def _tc2_body(acc2_ref, p_ref, r_ref, accc_ref, out_ref):
    accc = accc_ref[...]
    cnt = accc[0, :, IN_DIM] + accc[1, :, IN_DIM] + 1.0
    g2 = acc2_ref[0] + acc2_ref[1] + p_ref[...]
    z = g2 / cnt[:, None] + r_ref[...]
    m = jnp.max(z, axis=1, keepdims=True)
    e = jnp.exp(z - m)
    out_ref[...] = (z - m) - jnp.log(jnp.sum(e, axis=1, keepdims=True))


def kernel(x, edge_index, edge_attr, W1_l, b1_l, W1_r, W2_l, b2_l, W2_r):
    esrc = edge_index[0].reshape(NROWS, 1, CHUNK)
    edst = edge_index[1].reshape(NROWS, 1, CHUNK)
    ewb = lax.bitcast_convert_type(edge_attr, jnp.int32).reshape(
        NROWS, 1, CHUNK)
    edata = jnp.concatenate([esrc, edst, ewb], axis=1)

    acc1 = _sc_pass1(edata, x)

    nblk = N_NODES // BLK
    full = lambda *shape: pl.BlockSpec(shape, lambda i: (0,) * len(shape))
    p, r = pl.pallas_call(
        _tc1_body,
        grid=(nblk,),
        in_specs=[
            pl.BlockSpec((NC, BLK, AUG), lambda i: (0, i, 0)),
            pl.BlockSpec((BLK, IN_DIM), lambda i: (i, 0)),
            full(IN_DIM, HID_DIM),
            full(1, HID_DIM),
            full(IN_DIM, HID_DIM),
            full(HID_DIM, OUT_DIM),
            full(1, OUT_DIM),
            full(HID_DIM, OUT_DIM),
        ],
        out_specs=[
            pl.BlockSpec((BLK, OUT_DIM), lambda i: (i, 0)),
            pl.BlockSpec((BLK, OUT_DIM), lambda i: (i, 0)),
        ],
        out_shape=[
            jax.ShapeDtypeStruct((N_NODES, OUT_DIM), jnp.float32),
            jax.ShapeDtypeStruct((N_NODES, OUT_DIM), jnp.float32),
        ],
    )(acc1, x, W1_l, b1_l.reshape(1, HID_DIM), W1_r, W2_l,
      b2_l.reshape(1, OUT_DIM), W2_r)

    acc2 = _sc_pass2(edata, p)

    out = pl.pallas_call(
        _tc2_body,
        grid=(nblk,),
        in_specs=[
            pl.BlockSpec((NC, BLK, OUT_DIM), lambda i: (0, i, 0)),
            pl.BlockSpec((BLK, OUT_DIM), lambda i: (i, 0)),
            pl.BlockSpec((BLK, OUT_DIM), lambda i: (i, 0)),
            pl.BlockSpec((NC, BLK, AUG), lambda i: (0, i, 0)),
        ],
        out_specs=pl.BlockSpec((BLK, OUT_DIM), lambda i: (i, 0)),
        out_shape=jax.ShapeDtypeStruct((N_NODES, OUT_DIM), jnp.float32),
    )(acc2, p, r, acc1)

    return out


# speedup vs baseline: 12.9156x; 1.2104x over previous
import functools

import jax
import jax.numpy as jnp
from jax import lax
from jax.experimental import pallas as pl
from jax.experimental.pallas import tpu as pltpu
from jax.experimental.pallas import tpu_sc as plsc

N_NODES = 10000
N_EDGES = 320000
IN_DIM = 128
HID_DIM = 256
OUT_DIM = 64

NC, NS = 2, 16
NW = NC * NS
CHUNK = 64
NROWS = N_EDGES // CHUNK
CPT = NROWS // NW
XTRA = NROWS - CPT * NW
NPAD = 10112
RPT = NPAD // NS
HAF = IN_DIM // 2
AUG = 16

_mesh = plsc.VectorSubcoreMesh(core_axis_name="c", subcore_axis_name="s")


def _make_sc_pass(feat, aug):
    outw = feat + aug

    @functools.partial(
        pl.kernel,
        out_type=jax.ShapeDtypeStruct((NC, NPAD, outw), jnp.float32),
        mesh=_mesh,
        compiler_params=pltpu.CompilerParams(use_tc_tiling_on_sc=False,
                                             needs_layout_passes=False),
        scratch_types=[
            pltpu.VMEM((3, CHUNK), jnp.int32),
            pltpu.VMEM((3, CHUNK), jnp.int32),
            pltpu.VMEM((3, CHUNK), jnp.int32),
            pltpu.VMEM((3, CHUNK), jnp.int32),
            pltpu.VMEM((CHUNK, feat), jnp.float32),
            pltpu.VMEM((CHUNK, feat), jnp.float32),
            pltpu.VMEM((CHUNK, outw), jnp.float32),
            pltpu.VMEM((CHUNK, outw), jnp.float32),
            pltpu.VMEM_SHARED((NPAD, outw), jnp.float32),
            pltpu.SemaphoreType.DMA,
            pltpu.SemaphoreType.DMA,
            pltpu.SemaphoreType.DMA,
            pltpu.SemaphoreType.DMA,
            pltpu.SemaphoreType.DMA,
            pltpu.SemaphoreType.DMA,
            pltpu.SemaphoreType.DMA,
            pltpu.SemaphoreType.DMA,
        ],
    )
    def sc_pass(ed_hbm, tab_hbm, out_hbm,
                i0, i1, i2, i3, r0, r1, m0, m1, acc_sh,
                is0, is1, is2, is3, g0, g1, s0, s1):
        c = lax.axis_index("c")
        s = lax.axis_index("s")
        base_r = s * RPT
        ibuf, isem = (i0, i1, i2, i3), (is0, is1, is2, is3)
        rows, msgs, gs, ss = (r0, r1), (m0, m1), (g0, g1), (s0, s1)

        wid = c * NS + s
        start = wid * CPT + jnp.maximum(wid - (NW - XTRA), 0)
        has_xtra = wid >= NW - XTRA

        zero = jnp.zeros((16,), jnp.float32)
        one = jnp.ones((16,), jnp.float32)

        def zbody(i, _):
            for j in range(outw // 16):
                m0[i, pl.ds(j * 16, 16)] = zero
            return 0

        lax.fori_loop(0, CHUNK, zbody, 0)
        for k in range(RPT // CHUNK):
            pltpu.sync_copy(m0, acc_sh.at[pl.ds(base_r + k * CHUNK, CHUNK)])
        rem = RPT % CHUNK
        if rem:
            pltpu.sync_copy(
                m0.at[pl.ds(0, rem)],
                acc_sh.at[pl.ds(base_r + (RPT // CHUNK) * CHUNK, rem)])
        if aug:

            def obody(i, _):
                m0[i, pl.ds(feat, 16)] = one
                m1[i, pl.ds(feat, 16)] = one
                return 0

            lax.fori_loop(0, CHUNK, obody, 0)
        plsc.subcore_barrier()

        def idx_row(k):
            return jnp.minimum(start + k, NROWS - 1)

        def scale(j, b):
            def sbody(t, _):
                wvec = plsc.bitcast(ibuf[j][2, pl.ds(t * 16, 16)], jnp.float32)
                for l in range(16):
                    we = wvec[l]
                    row = t * 16 + l
                    for v in range(feat // 16):
                        msgs[b][row, pl.ds(v * 16, 16)] = (
                            rows[b][row, pl.ds(v * 16, 16)] * we)
                return 0

            lax.fori_loop(0, CHUNK // 16, sbody, 0)

        pltpu.async_copy(ed_hbm.at[idx_row(0)], ibuf[0], isem[0])
        pltpu.async_copy(ed_hbm.at[idx_row(1)], ibuf[1], isem[1])
        pltpu.make_async_copy(ed_hbm.at[0], ibuf[0], isem[0]).wait()
        pltpu.async_copy(tab_hbm.at[ibuf[0].at[0]], rows[0], gs[0])

        def body(i, _):
            for b in range(4):
                b2, nb2 = b % 2, (b + 1) % 2
                j2, j1 = (b + 2) % 4, (b + 1) % 4
                k = i * 4 + b
                if b < 2:
                    @pl.when(i > 0)
                    def _():
                        pltpu.make_async_copy(
                            msgs[b2], acc_sh.at[ibuf[b].at[1]], ss[b2]).wait()
                else:
                    pltpu.make_async_copy(
                        msgs[b2], acc_sh.at[ibuf[b].at[1]], ss[b2]).wait()
                pltpu.async_copy(ed_hbm.at[idx_row(k + 2)], ibuf[j2],
                                 isem[j2])
                pltpu.make_async_copy(ed_hbm.at[0], ibuf[j1], isem[j1]).wait()
                pltpu.async_copy(tab_hbm.at[ibuf[j1].at[0]], rows[nb2],
                                 gs[nb2])
                pltpu.make_async_copy(tab_hbm.at[ibuf[b].at[0]], rows[b2],
                                      gs[b2]).wait()
                scale(b, b2)
                pltpu.async_copy(msgs[b2], acc_sh.at[ibuf[b].at[1]], ss[b2],
                                 add=True)
            return 0

        lax.fori_loop(0, CPT // 4, body, 0)

        pltpu.make_async_copy(ed_hbm.at[0], ibuf[1], isem[1]).wait()
        pltpu.make_async_copy(tab_hbm.at[ibuf[0].at[0]], rows[0], gs[0]).wait()
        pltpu.make_async_copy(msgs[0], acc_sh.at[ibuf[0].at[1]], ss[0]).wait()
        pltpu.make_async_copy(msgs[1], acc_sh.at[ibuf[1].at[1]], ss[1]).wait()

        @pl.when(has_xtra)
        def _():
            scale(0, 0)
            pltpu.sync_copy(msgs[0], acc_sh.at[ibuf[0].at[1]], add=True)

        plsc.subcore_barrier()
        pltpu.sync_copy(acc_sh.at[pl.ds(base_r, RPT)],
                        out_hbm.at[c, pl.ds(base_r, RPT)])

    return sc_pass


_sc_pass1a = _make_sc_pass(HAF, AUG)
_sc_pass1b = _make_sc_pass(HAF, 0)
_sc_pass2 = _make_sc_pass(OUT_DIM, 0)


BLK = 1000


def _tc1_body(acca_ref, accb_ref, x_ref, w1l_ref, b1l_ref, w1r_ref, w2l_ref,
              b2l_ref, w2r_ref, p_ref, r_ref):
    acca = acca_ref[0] + acca_ref[1]
    accb = accb_ref[0] + accb_ref[1]
    x = x_ref[...]
    agg = jnp.concatenate([acca[:, :HAF], accb], axis=1) + x
    cnt = acca[:, HAF] + 1.0
    agg = agg / cnt[:, None]
    h = agg @ w1l_ref[...] + b1l_ref[...] + x @ w1r_ref[...]
    h = jnp.maximum(h, 0.0)
    p_ref[...] = h @ w2l_ref[...]
    r_ref[...] = h @ w2r_ref[...] + b2l_ref[...]


def _tc2_body(acc2_ref, p_ref, r_ref, accc_ref, out_ref):
    accc = accc_ref[...]
    cnt = accc[0, :, HAF] + accc[1, :, HAF] + 1.0
    g2 = acc2_ref[0] + acc2_ref[1] + p_ref[...]
    z = g2 / cnt[:, None] + r_ref[...]
    m = jnp.max(z, axis=1, keepdims=True)
    e = jnp.exp(z - m)
    out_ref[...] = (z - m) - jnp.log(jnp.sum(e, axis=1, keepdims=True))


def kernel(x, edge_index, edge_attr, W1_l, b1_l, W1_r, W2_l, b2_l, W2_r):
    esrc = edge_index[0].reshape(NROWS, 1, CHUNK)
    edst = edge_index[1].reshape(NROWS, 1, CHUNK)
    ewb = lax.bitcast_convert_type(edge_attr, jnp.int32).reshape(
        NROWS, 1, CHUNK)
    edata = jnp.concatenate([esrc, edst, ewb], axis=1)

    xa = x[:, :HAF]
    xb = x[:, HAF:]
    acc1a = _sc_pass1a(edata, xa)
    acc1b = _sc_pass1b(edata, xb)

    nblk = N_NODES // BLK
    full = lambda *shape: pl.BlockSpec(shape, lambda i: (0,) * len(shape))
    p, r = pl.pallas_call(
        _tc1_body,
        grid=(nblk,),
        in_specs=[
            pl.BlockSpec((NC, BLK, HAF + AUG), lambda i: (0, i, 0)),
            pl.BlockSpec((NC, BLK, HAF), lambda i: (0, i, 0)),
            pl.BlockSpec((BLK, IN_DIM), lambda i: (i, 0)),
            full(IN_DIM, HID_DIM),
            full(1, HID_DIM),
            full(IN_DIM, HID_DIM),
            full(HID_DIM, OUT_DIM),
            full(1, OUT_DIM),
            full(HID_DIM, OUT_DIM),
        ],
        out_specs=[
            pl.BlockSpec((BLK, OUT_DIM), lambda i: (i, 0)),
            pl.BlockSpec((BLK, OUT_DIM), lambda i: (i, 0)),
        ],
        out_shape=[
            jax.ShapeDtypeStruct((N_NODES, OUT_DIM), jnp.float32),
            jax.ShapeDtypeStruct((N_NODES, OUT_DIM), jnp.float32),
        ],
    )(acc1a, acc1b, x, W1_l, b1_l.reshape(1, HID_DIM), W1_r, W2_l,
      b2_l.reshape(1, OUT_DIM), W2_r)

    acc2 = _sc_pass2(edata, p)

    out = pl.pallas_call(
        _tc2_body,
        grid=(nblk,),
        in_specs=[
            pl.BlockSpec((NC, BLK, OUT_DIM), lambda i: (0, i, 0)),
            pl.BlockSpec((BLK, OUT_DIM), lambda i: (i, 0)),
            pl.BlockSpec((BLK, OUT_DIM), lambda i: (i, 0)),
            pl.BlockSpec((NC, BLK, HAF + AUG), lambda i: (0, i, 0)),
        ],
        out_specs=pl.BlockSpec((BLK, OUT_DIM), lambda i: (i, 0)),
        out_shape=jax.ShapeDtypeStruct((N_NODES, OUT_DIM), jnp.float32),
    )(acc2, p, r, acc1a)

    return out
